# baseline (speedup 1.0000x reference)
import functools

import jax
import jax.numpy as jnp
from jax import lax
from jax.experimental import pallas as pl
from jax.experimental.pallas import tpu as pltpu
from jax.experimental.pallas import tpu_sc as plsc

NC, NS, LANES = 2, 16, 16
NW = NC * NS
CHUNK = 128


def _mm1_body(x_ref, w_ref, o_ref):
    o_ref[...] = jnp.dot(x_ref[...], w_ref[...],
                         preferred_element_type=jnp.float32)


def _mm2_body(a_ref, b_ref, w_ref, o_ref):
    h = jnp.maximum(a_ref[...] + b_ref[...], 0.0)
    o_ref[...] = jnp.dot(h, w_ref[...], preferred_element_type=jnp.float32)


def _matmul1(x, W, bn=1000):
    N, D = x.shape
    return pl.pallas_call(
        _mm1_body,
        grid=(N // bn,),
        in_specs=[pl.BlockSpec((bn, D), lambda i: (i, 0)),
                  pl.BlockSpec((D, D), lambda i: (0, 0))],
        out_specs=pl.BlockSpec((bn, D), lambda i: (i, 0)),
        out_shape=jax.ShapeDtypeStruct((N, D), jnp.float32),
    )(x, W)


def _matmul2(a, b, W, N, bn=1000):
    D = a.shape[1]
    return pl.pallas_call(
        _mm2_body,
        grid=(N // bn,),
        in_specs=[pl.BlockSpec((bn, D), lambda i: (i, 0)),
                  pl.BlockSpec((bn, D), lambda i: (i, 0)),
                  pl.BlockSpec((D, D), lambda i: (0, 0))],
        out_specs=pl.BlockSpec((bn, D), lambda i: (i, 0)),
        out_shape=jax.ShapeDtypeStruct((N, D), jnp.float32),
    )(a, b, W)


def _spmm(support, rowsc, colsc, valsc, NP, codeid=None):
    N, D = support.shape
    nchunks = rowsc.shape[0]
    per_pair = nchunks // NS
    PT0 = max(16, int(round(per_pair * 0.71 / 16)) * 16)
    PT1 = per_pair - PT0
    rows_per_tile = NP // NS
    final_gather = codeid is not None
    if final_gather:
        B = codeid.shape[0] * codeid.shape[1] * codeid.shape[2]
        id_chunks = codeid.shape[1]
        out_type = [jax.ShapeDtypeStruct((B, D), jnp.float32)] * 2
    else:
        out_type = [jax.ShapeDtypeStruct((NP, D), jnp.float32)] * 2

    SEG = 32
    scratch = [
        pltpu.VMEM((SEG, CHUNK), jnp.int32),
        pltpu.VMEM((SEG, CHUNK), jnp.int32),
        pltpu.VMEM((SEG, CHUNK), jnp.float32),
        pltpu.VMEM((CHUNK, D), jnp.float32),
        pltpu.VMEM((CHUNK, D), jnp.float32),
        pltpu.VMEM_SHARED((NP, D), jnp.float32),
        pltpu.SemaphoreType.DMA,
        pltpu.SemaphoreType.DMA,
        pltpu.SemaphoreType.DMA,
        pltpu.SemaphoreType.DMA,
    ]
    if final_gather:
        scratch.append(pltpu.VMEM((id_chunks, CHUNK), jnp.int32))

    mesh = plsc.VectorSubcoreMesh(core_axis_name="c", subcore_axis_name="s")

    def body(sup_hbm, rows_hbm, cols_hbm, vals_hbm, *rest):
        if final_gather:
            (id_hbm, out0, out1, colv, rowv, valv, g0, g1, acc,
             sg0, sg1, ss0, ss1, idv) = rest
        else:
            (out0, out1, colv, rowv, valv, g0, g1, acc,
             sg0, sg1, ss0, ss1) = rest
        gbufs = (g0, g1)
        sems = (sg0, sg1)
        c = lax.axis_index("c")
        s = lax.axis_index("s")
        wid = c * NS + s
        r0 = s * rows_per_tile
        zv = jnp.zeros((LANES,), jnp.float32)

        def zrow(i, carry):
            for q in range(D // LANES):
                g0[i, pl.ds(q * LANES, LANES)] = zv
            return carry

        lax.fori_loop(0, CHUNK, zrow, 0, unroll=False)
        off0 = 0
        while off0 < rows_per_tile:
            zl = min(CHUNK, rows_per_tile - off0)
            pltpu.sync_copy(g0.at[pl.ds(0, zl)],
                            acc.at[pl.ds(r0 + off0, zl)])
            off0 += zl
        plsc.subcore_barrier()

        def do_chunk(j, gb, sg, ss, gnext, sgnext, ssnext, lim):
            pltpu.make_async_copy(sup_hbm.at[colv.at[j]], gb, sg).wait()

            @pl.when(j >= 1)
            def _():
                pltpu.make_async_copy(gnext, acc.at[rowv.at[0]],
                                      ssnext).wait()

            @pl.when(j + 1 < lim)
            def _():
                pltpu.async_copy(sup_hbm.at[colv.at[j + 1]], gnext, sgnext)

            def edge_group(g, carry2):
                vv = valv[j, pl.ds(g * LANES, LANES)]
                for t in range(LANES):
                    i = g * LANES + t
                    vb = jnp.full((LANES,), vv[t], jnp.float32)
                    for q in range(D // LANES):
                        sl = pl.ds(q * LANES, LANES)
                        gb[i, sl] = gb[i, sl] * vb
                return carry2

            lax.fori_loop(0, CHUNK // LANES, edge_group, 0, unroll=False)
            pltpu.async_copy(gb, acc.at[rowv.at[j]], ss, add=True)

        def run_edges(base, pt):
            off = 0
            while off < pt:
                sl = min(SEG, pt - off)

                def chunk_pair(jj, carry, sl=sl):
                    do_chunk(2 * jj, g0, sg0, ss0, g1, sg1, ss1, sl)
                    do_chunk(2 * jj + 1, g1, sg1, ss1, g0, sg0, ss0, sl)
                    return carry

                sb = base + off
                pltpu.sync_copy(cols_hbm.at[pl.ds(sb, sl)],
                                colv.at[pl.ds(0, sl)])
                pltpu.sync_copy(rows_hbm.at[pl.ds(sb, sl)],
                                rowv.at[pl.ds(0, sl)])
                pltpu.sync_copy(vals_hbm.at[pl.ds(sb, sl)],
                                valv.at[pl.ds(0, sl)])
                pltpu.async_copy(sup_hbm.at[colv.at[0]], g0, sg0)
                lax.fori_loop(0, sl // 2, chunk_pair, 0, unroll=False)
                pltpu.make_async_copy(g1, acc.at[rowv.at[0]], ss1).wait()
                off += sl

        @pl.when(c == 0)
        def _():
            run_edges(s * PT0, PT0)

        @pl.when(c == 1)
        def _():
            run_edges(NS * PT0 + s * PT1, PT1)
        plsc.subcore_barrier()

        if final_gather:
            pltpu.sync_copy(id_hbm.at[s], idv)
            for k in range(id_chunks):
                gb, sm = gbufs[k % 2], sems[k % 2]
                pltpu.async_copy(acc.at[idv.at[k]], gb, sm).wait()
                ob = s * id_chunks * CHUNK + k * CHUNK

                @pl.when(c == 0)
                def _():
                    pltpu.sync_copy(gb, out0.at[pl.ds(ob, CHUNK)])

                @pl.when(c == 1)
                def _():
                    pltpu.sync_copy(gb, out1.at[pl.ds(ob, CHUNK)])
        else:
            @pl.when(c == 0)
            def _():
                pltpu.sync_copy(acc.at[pl.ds(r0, rows_per_tile)],
                                out0.at[pl.ds(r0, rows_per_tile)])

            @pl.when(c == 1)
            def _():
                pltpu.sync_copy(acc.at[pl.ds(r0, rows_per_tile)],
                                out1.at[pl.ds(r0, rows_per_tile)])

    k = pl.kernel(body, out_type=out_type, mesh=mesh, scratch_types=scratch)
    if final_gather:
        return k(support, rowsc, colsc, valsc, codeid)
    return k(support, rowsc, colsc, valsc)


def _rnn_body(pid_ref, ce0_ref, ce1_ref, td_ref, f_ref, pd_ref,
              wih_ref, whh_ref, bih_ref, bhh_ref, o_ref):
    D = ce0_ref.shape[1]
    ce = jnp.maximum(ce0_ref[...] + ce1_ref[...], 0.0)
    sub = pid_ref[0] - (pid_ref[0] // 8) * 8
    rows8 = lax.broadcasted_iota(jnp.int32, (8, 1), 0)
    sel = jnp.where(rows8 == sub, 1.0, 0.0)
    pe = jnp.sum(pd_ref[...] * sel, axis=0, keepdims=True)
    wih = wih_ref[...]
    dn = (((1,), (1,)), ((), ()))
    row = lax.dot_general(pe, wih[:, :D], dn,
                          preferred_element_type=jnp.float32)
    wt = wih[:, D:D + 1]
    m = (row + td_ref[...] * wt.T
         + lax.dot_general(f_ref[...], wih[:, D + 1:], dn,
                           preferred_element_type=jnp.float32)
         + lax.dot_general(ce, whh_ref[...], dn,
                           preferred_element_type=jnp.float32)
         + bih_ref[...] + bhh_ref[...])
    h = jnp.tanh(m)
    ss = jnp.sum(h * h, axis=1, keepdims=True)
    o_ref[...] = h / jnp.maximum(jnp.sqrt(ss), 1e-12)


def _rnn(ce0, ce1, timediffs, features, patient_dynamic, pid,
         W_ih, W_hh, b_ih, b_hh, bn=512):
    B, D = ce0.shape
    F = features.shape[1]
    IN = W_ih.shape[1]
    grid_spec = pltpu.PrefetchScalarGridSpec(
        num_scalar_prefetch=1,
        grid=(B // bn,),
        in_specs=[
            pl.BlockSpec((bn, D), lambda i, pid: (i, 0)),
            pl.BlockSpec((bn, D), lambda i, pid: (i, 0)),
            pl.BlockSpec((bn, 1), lambda i, pid: (i, 0)),
            pl.BlockSpec((bn, F), lambda i, pid: (i, 0)),
            pl.BlockSpec((8, D), lambda i, pid: (pid[0] // 8, 0)),
            pl.BlockSpec((D, IN), lambda i, pid: (0, 0)),
            pl.BlockSpec((D, D), lambda i, pid: (0, 0)),
            pl.BlockSpec((1, D), lambda i, pid: (0, 0)),
            pl.BlockSpec((1, D), lambda i, pid: (0, 0)),
        ],
        out_specs=pl.BlockSpec((bn, D), lambda i, pid: (i, 0)),
    )
    return pl.pallas_call(
        _rnn_body,
        grid_spec=grid_spec,
        out_shape=jax.ShapeDtypeStruct((B, D), jnp.float32),
    )(pid, ce0, ce1, timediffs, features, patient_dynamic,
      W_ih, W_hh, b_ih.reshape(1, D), b_hh.reshape(1, D))


def kernel(patient_dynamic, code_dynamic, init_code_dynamic, adj_indices,
           adj_values, patientid, codeid, ancestorid, features, timediffs,
           W1, W2, W_ih, W_hh, b_ih, b_hh):
    N, D = code_dynamic.shape
    E = adj_values.shape[0]
    nchunks = -(-E // (NW * 8 * CHUNK)) * NW * 8
    padE = nchunks * CHUNK - E
    rows = jnp.concatenate([adj_indices[0], jnp.zeros((padE,), jnp.int32)])
    cols = jnp.concatenate([adj_indices[1], jnp.zeros((padE,), jnp.int32)])
    vals = jnp.concatenate([adj_values, jnp.zeros((padE,), jnp.float32)])
    rowsc = rows.reshape(nchunks, CHUNK)
    colsc = cols.reshape(nchunks, CHUNK)
    valsc = vals.reshape(nchunks, CHUNK)
    NP = -(-N // (NS * 8)) * NS * 8
    pid = jnp.asarray(patientid, jnp.int32).reshape(1)
    B = codeid.shape[0]
    cid = jnp.asarray(codeid, jnp.int32).reshape(NS, B // NS // CHUNK, CHUNK)

    support1 = _matmul1(code_dynamic, W1)
    h1a, h1b = _spmm(support1, rowsc, colsc, valsc, NP)
    support2 = _matmul2(h1a, h1b, W2, N)
    ce0, ce1 = _spmm(support2, rowsc, colsc, valsc, NP, codeid=cid)
    return _rnn(ce0, ce1, timediffs, features, patient_dynamic, pid,
                W_ih, W_hh, b_ih, b_hh)


# speedup vs baseline: 1.0230x; 1.0230x over previous
import functools

import jax
import jax.numpy as jnp
from jax import lax
from jax.experimental import pallas as pl
from jax.experimental.pallas import tpu as pltpu
from jax.experimental.pallas import tpu_sc as plsc

NC, NS, LANES = 2, 16, 16
NW = NC * NS
CHUNK = 128


def _mm1_body(x_ref, w_ref, o_ref):
    o_ref[...] = jnp.dot(x_ref[...], w_ref[...],
                         preferred_element_type=jnp.float32)


def _mm2_body(a_ref, b_ref, w_ref, o_ref):
    h = jnp.maximum(a_ref[...] + b_ref[...], 0.0)
    o_ref[...] = jnp.dot(h, w_ref[...], preferred_element_type=jnp.float32)


def _matmul1(x, W, bn=1000):
    N, D = x.shape
    return pl.pallas_call(
        _mm1_body,
        grid=(N // bn,),
        in_specs=[pl.BlockSpec((bn, D), lambda i: (i, 0)),
                  pl.BlockSpec((D, D), lambda i: (0, 0))],
        out_specs=pl.BlockSpec((bn, D), lambda i: (i, 0)),
        out_shape=jax.ShapeDtypeStruct((N, D), jnp.float32),
    )(x, W)


def _matmul2(a, b, W, N, bn=1000):
    D = a.shape[1]
    return pl.pallas_call(
        _mm2_body,
        grid=(N // bn,),
        in_specs=[pl.BlockSpec((bn, D), lambda i: (i, 0)),
                  pl.BlockSpec((bn, D), lambda i: (i, 0)),
                  pl.BlockSpec((D, D), lambda i: (0, 0))],
        out_specs=pl.BlockSpec((bn, D), lambda i: (i, 0)),
        out_shape=jax.ShapeDtypeStruct((N, D), jnp.float32),
    )(a, b, W)


def _spmm(support, rowsc, colsc, valsc, NP, codeid=None):
    N, D = support.shape
    nchunks = rowsc.shape[0]
    per_pair = nchunks // NS
    PT0 = max(16, int(round(per_pair * 0.90 / 16)) * 16)
    PT1 = per_pair - PT0
    rows_per_tile = NP // NS
    final_gather = codeid is not None
    if final_gather:
        B = codeid.shape[0] * codeid.shape[1] * codeid.shape[2]
        id_chunks = codeid.shape[1]
        out_type = [jax.ShapeDtypeStruct((B, D), jnp.float32)] * 2
    else:
        out_type = [jax.ShapeDtypeStruct((NP, D), jnp.float32)] * 2

    SEG = 32
    scratch = [
        pltpu.VMEM((SEG, CHUNK), jnp.int32),
        pltpu.VMEM((SEG, CHUNK), jnp.int32),
        pltpu.VMEM((SEG, CHUNK), jnp.float32),
        pltpu.VMEM((CHUNK, D), jnp.float32),
        pltpu.VMEM((CHUNK, D), jnp.float32),
        pltpu.VMEM_SHARED((NP, D), jnp.float32),
        pltpu.SemaphoreType.DMA,
        pltpu.SemaphoreType.DMA,
    ]
    if final_gather:
        scratch.append(pltpu.VMEM((id_chunks, CHUNK), jnp.int32))

    mesh = plsc.VectorSubcoreMesh(core_axis_name="c", subcore_axis_name="s")

    def body(sup_hbm, rows_hbm, cols_hbm, vals_hbm, *rest):
        if final_gather:
            (id_hbm, out0, out1, colv, rowv, valv, g0, g1, acc,
             sg0, sg1, idv) = rest
        else:
            (out0, out1, colv, rowv, valv, g0, g1, acc,
             sg0, sg1) = rest
        c = lax.axis_index("c")
        s = lax.axis_index("s")
        wid = c * NS + s
        r0 = s * rows_per_tile
        zv = jnp.zeros((LANES,), jnp.float32)

        def zrow(i, carry):
            for q in range(D // LANES):
                g0[i, pl.ds(q * LANES, LANES)] = zv
            return carry

        lax.fori_loop(0, CHUNK, zrow, 0, unroll=False)
        off0 = 0
        while off0 < rows_per_tile:
            zl = min(CHUNK, rows_per_tile - off0)
            pltpu.sync_copy(g0.at[pl.ds(0, zl)],
                            acc.at[pl.ds(r0 + off0, zl)])
            off0 += zl
        plsc.subcore_barrier()

        def do_chunk(j, gb, sg, gnext, sgnext, lim):
            pltpu.make_async_copy(sup_hbm.at[colv.at[j]], gb, sg).wait()

            @pl.when(j + 1 < lim)
            def _():
                pltpu.async_copy(sup_hbm.at[colv.at[j + 1]], gnext, sgnext)

            def edge_group(g, carry2):
                vv = valv[j, pl.ds(g * LANES, LANES)]
                for t in range(LANES):
                    i = g * LANES + t
                    vb = jnp.full((LANES,), vv[t], jnp.float32)
                    for q in range(D // LANES):
                        sl = pl.ds(q * LANES, LANES)
                        gb[i, sl] = gb[i, sl] * vb
                return carry2

            lax.fori_loop(0, CHUNK // LANES, edge_group, 0, unroll=False)
            pltpu.sync_copy(gb, acc.at[rowv.at[j]], add=True)

        def run_edges(base, pt):
            off = 0
            while off < pt:
                sl = min(SEG, pt - off)

                def chunk_pair(jj, carry, sl=sl):
                    do_chunk(2 * jj, g0, sg0, g1, sg1, sl)
                    do_chunk(2 * jj + 1, g1, sg1, g0, sg0, sl)
                    return carry

                sb = base + off
                pltpu.sync_copy(cols_hbm.at[pl.ds(sb, sl)],
                                colv.at[pl.ds(0, sl)])
                pltpu.sync_copy(rows_hbm.at[pl.ds(sb, sl)],
                                rowv.at[pl.ds(0, sl)])
                pltpu.sync_copy(vals_hbm.at[pl.ds(sb, sl)],
                                valv.at[pl.ds(0, sl)])
                pltpu.async_copy(sup_hbm.at[colv.at[0]], g0, sg0)
                lax.fori_loop(0, sl // 2, chunk_pair, 0, unroll=False)
                off += sl

        @pl.when(c == 0)
        def _():
            run_edges(s * PT0, PT0)

        @pl.when(c == 1)
        def _():
            run_edges(NS * PT0 + s * PT1, PT1)
        plsc.subcore_barrier()

        if final_gather:
            pltpu.sync_copy(id_hbm.at[s], idv)
            for k in range(id_chunks):
                gb, sm = (g0, sg0) if k % 2 == 0 else (g1, sg1)
                pltpu.async_copy(acc.at[idv.at[k]], gb, sm).wait()
                ob = s * id_chunks * CHUNK + k * CHUNK

                @pl.when(c == 0)
                def _():
                    pltpu.sync_copy(gb, out0.at[pl.ds(ob, CHUNK)])

                @pl.when(c == 1)
                def _():
                    pltpu.sync_copy(gb, out1.at[pl.ds(ob, CHUNK)])
        else:
            @pl.when(c == 0)
            def _():
                pltpu.sync_copy(acc.at[pl.ds(r0, rows_per_tile)],
                                out0.at[pl.ds(r0, rows_per_tile)])

            @pl.when(c == 1)
            def _():
                pltpu.sync_copy(acc.at[pl.ds(r0, rows_per_tile)],
                                out1.at[pl.ds(r0, rows_per_tile)])

    k = pl.kernel(body, out_type=out_type, mesh=mesh, scratch_types=scratch)
    if final_gather:
        return k(support, rowsc, colsc, valsc, codeid)
    return k(support, rowsc, colsc, valsc)


def _rnn_body(pid_ref, ce0_ref, ce1_ref, td_ref, f_ref, pd_ref,
              wih_ref, whh_ref, bih_ref, bhh_ref, o_ref):
    D = ce0_ref.shape[1]
    ce = jnp.maximum(ce0_ref[...] + ce1_ref[...], 0.0)
    sub = pid_ref[0] - (pid_ref[0] // 8) * 8
    rows8 = lax.broadcasted_iota(jnp.int32, (8, 1), 0)
    sel = jnp.where(rows8 == sub, 1.0, 0.0)
    pe = jnp.sum(pd_ref[...] * sel, axis=0, keepdims=True)
    wih = wih_ref[...]
    dn = (((1,), (1,)), ((), ()))
    row = lax.dot_general(pe, wih[:, :D], dn,
                          preferred_element_type=jnp.float32)
    wt = wih[:, D:D + 1]
    m = (row + td_ref[...] * wt.T
         + lax.dot_general(f_ref[...], wih[:, D + 1:], dn,
                           preferred_element_type=jnp.float32)
         + lax.dot_general(ce, whh_ref[...], dn,
                           preferred_element_type=jnp.float32)
         + bih_ref[...] + bhh_ref[...])
    h = jnp.tanh(m)
    ss = jnp.sum(h * h, axis=1, keepdims=True)
    o_ref[...] = h / jnp.maximum(jnp.sqrt(ss), 1e-12)


def _rnn(ce0, ce1, timediffs, features, patient_dynamic, pid,
         W_ih, W_hh, b_ih, b_hh, bn=512):
    B, D = ce0.shape
    F = features.shape[1]
    IN = W_ih.shape[1]
    grid_spec = pltpu.PrefetchScalarGridSpec(
        num_scalar_prefetch=1,
        grid=(B // bn,),
        in_specs=[
            pl.BlockSpec((bn, D), lambda i, pid: (i, 0)),
            pl.BlockSpec((bn, D), lambda i, pid: (i, 0)),
            pl.BlockSpec((bn, 1), lambda i, pid: (i, 0)),
            pl.BlockSpec((bn, F), lambda i, pid: (i, 0)),
            pl.BlockSpec((8, D), lambda i, pid: (pid[0] // 8, 0)),
            pl.BlockSpec((D, IN), lambda i, pid: (0, 0)),
            pl.BlockSpec((D, D), lambda i, pid: (0, 0)),
            pl.BlockSpec((1, D), lambda i, pid: (0, 0)),
            pl.BlockSpec((1, D), lambda i, pid: (0, 0)),
        ],
        out_specs=pl.BlockSpec((bn, D), lambda i, pid: (i, 0)),
    )
    return pl.pallas_call(
        _rnn_body,
        grid_spec=grid_spec,
        out_shape=jax.ShapeDtypeStruct((B, D), jnp.float32),
    )(pid, ce0, ce1, timediffs, features, patient_dynamic,
      W_ih, W_hh, b_ih.reshape(1, D), b_hh.reshape(1, D))


def kernel(patient_dynamic, code_dynamic, init_code_dynamic, adj_indices,
           adj_values, patientid, codeid, ancestorid, features, timediffs,
           W1, W2, W_ih, W_hh, b_ih, b_hh):
    N, D = code_dynamic.shape
    E = adj_values.shape[0]
    nchunks = -(-E // (NW * 8 * CHUNK)) * NW * 8
    padE = nchunks * CHUNK - E
    rows = jnp.concatenate([adj_indices[0], jnp.zeros((padE,), jnp.int32)])
    cols = jnp.concatenate([adj_indices[1], jnp.zeros((padE,), jnp.int32)])
    vals = jnp.concatenate([adj_values, jnp.zeros((padE,), jnp.float32)])
    rowsc = rows.reshape(nchunks, CHUNK)
    colsc = cols.reshape(nchunks, CHUNK)
    valsc = vals.reshape(nchunks, CHUNK)
    NP = -(-N // (NS * 8)) * NS * 8

    pid = jnp.asarray(patientid, jnp.int32).reshape(1)
    B = codeid.shape[0]
    cid = jnp.asarray(codeid, jnp.int32).reshape(NS, B // NS // CHUNK, CHUNK)

    support1 = _matmul1(code_dynamic, W1)
    h1a, h1b = _spmm(support1, rowsc, colsc, valsc, NP)
    support2 = _matmul2(h1a, h1b, W2, N)
    ce0, ce1 = _spmm(support2, rowsc, colsc, valsc, NP, codeid=cid)
    return _rnn(ce0, ce1, timediffs, features, patient_dynamic, pid,
                W_ih, W_hh, b_ih, b_hh)
